# baseline (speedup 1.0000x reference)
import jax
import jax.numpy as jnp
import numpy as np
from jax import lax
from jax.experimental import pallas as pl
from jax.experimental.pallas import tpu as pltpu

B = 16
C = 32
N = 32
J = 64
CHUNK = 512
K = J * CHUNK

def _gumbel_cbn():
    g = jax.random.gumbel(jax.random.key(123), (B, C, N), jnp.float32)
    return jnp.transpose(g, (1, 0, 2))


def _gather_body(filt_ref, x_ref, xs_ref, s_ref, acc):
    j = pl.program_id(0)

    @pl.when(j == 0)
    def _():
        acc[...] = jnp.zeros_like(acc)

    xb = x_ref[:, 0, 0, :]
    xs_ref[:, 0, 0, :] = xb
    acc[...] += jnp.sum(xb, axis=1, keepdims=True)

    @pl.when(j == J - 1)
    def _():
        s_ref[...] = acc[...]


def _main_body(s_ref, xs_ref, w_ref, g_ref, o_ref, xq, h3):
    c = pl.program_id(0)

    @pl.when(c == 0)
    def _():
        s = s_ref[...]
        xn = jnp.where(s > 0.0, xs_ref[...] / s, 0.0)
        xq[...] = jnp.transpose(xn.reshape(B, K // 8, 8), (2, 0, 1)).reshape(8 * B, K // 8)

    res = jnp.dot(xq[...], w_ref[0], preferred_element_type=jnp.float32)
    h3[c, :, :] = sum(
        lax.slice(res, (16 * q, 32 * q), (16 * q + 16, 32 * q + 32))
        for q in range(8)
    )

    @pl.when(c == C - 1)
    def _():
        total = jnp.float32(0.0)
        for cc in range(C):
            total += jnp.sum(jnp.max(h3[cc, :, :], axis=1))
        avg = total / jnp.float32(B * C)
        temp = 1.0 / (avg + jnp.float32(0.0001)) - 1.0
        iota2 = lax.broadcasted_iota(jnp.int32, (B, N), 1)
        for cc in range(C):
            z = h3[cc, :, :] / temp + g_ref[cc, :, :]
            m = jnp.max(z, axis=1, keepdims=True)
            cand = jnp.where(z == m, iota2, N)
            am = jnp.min(cand, axis=1, keepdims=True)
            o_ref[:, cc, :] = (iota2 == am).astype(jnp.int32)


def kernel(x, weights, input_filter):
    x4 = x.reshape(B, 1024, 1, CHUNK)
    g3 = _gumbel_cbn()

    xs4, row_sums = pl.pallas_call(
        _gather_body,
        grid_spec=pltpu.PrefetchScalarGridSpec(
            num_scalar_prefetch=1,
            grid=(J,),
            in_specs=[
                pl.BlockSpec((B, 1, 1, CHUNK),
                             lambda j, filt: (0, filt[j], 0, 0)),
            ],
            out_specs=[
                pl.BlockSpec((B, 1, 1, CHUNK), lambda j, filt: (0, j, 0, 0)),
                pl.BlockSpec((B, 1), lambda j, filt: (0, 0)),
            ],
            scratch_shapes=[pltpu.VMEM((B, 1), jnp.float32)],
        ),
        out_shape=[
            jax.ShapeDtypeStruct((B, J, 1, CHUNK), jnp.float32),
            jax.ShapeDtypeStruct((B, 1), jnp.float32),
        ],
    )(input_filter, x4)

    xs2 = xs4.reshape(B, K)

    out = pl.pallas_call(
        _main_body,
        grid=(C,),
        in_specs=[
            pl.BlockSpec((B, 1), lambda c: (0, 0)),
            pl.BlockSpec((B, K), lambda c: (0, 0)),
            pl.BlockSpec((1, K // 8, 8 * N), lambda c: (c, 0, 0)),
            pl.BlockSpec((C, B, N), lambda c: (0, 0, 0)),
        ],
        out_specs=pl.BlockSpec((B, C, N), lambda c: (0, 0, 0)),
        scratch_shapes=[
            pltpu.VMEM((8 * B, K // 8), jnp.float32),
            pltpu.VMEM((C, B, N), jnp.float32),
        ],
        out_shape=jax.ShapeDtypeStruct((B, C, N), jnp.int32),
    )(row_sums, xs2, weights.reshape(C, K // 8, 8 * N), g3)

    return out


# speedup vs baseline: 3.3186x; 3.3186x over previous
import jax
import jax.numpy as jnp
import numpy as np
from jax import lax
from jax.experimental import pallas as pl
from jax.experimental.pallas import tpu as pltpu

B = 16
C = 32
N = 32
J = 64
CHUNK = 512
K = J * CHUNK
BK = 1024
T = K // BK


def _gumbel_cnb():
    g = jax.random.gumbel(jax.random.key(123), (B, C, N), jnp.float32)
    return jnp.transpose(g, (1, 2, 0))


def _gather_body(filt_ref, x_ref, xs_ref, s_ref, acc):
    j = pl.program_id(0)

    @pl.when(j == 0)
    def _():
        acc[...] = jnp.zeros_like(acc)

    xb = x_ref[:, 0, 0, :]
    xs_ref[:, 0, 0, :] = xb
    acc[...] += jnp.sum(xb, axis=1, keepdims=True)

    @pl.when(j == J - 1)
    def _():
        s_ref[...] = acc[...]


def _main_body(s_ref, xs_ref, w_ref, g_ref, o_ref, ht):
    t = pl.program_id(0)

    @pl.when(t == 0)
    def _():
        ht[...] = jnp.zeros_like(ht)

    s = s_ref[...]
    xn = jnp.where(s > 0.0, xs_ref[...] / s, 0.0)
    wt2 = w_ref[...].reshape(C * N, BK)
    ht[...] += lax.dot_general(
        wt2, xn, (((1,), (1,)), ((), ())),
        preferred_element_type=jnp.float32)

    @pl.when(t == T - 1)
    def _():
        total = jnp.float32(0.0)
        for c in range(C):
            total += jnp.sum(jnp.max(ht[c * N:(c + 1) * N, :], axis=0))
        avg = total / jnp.float32(B * C)
        temp = 1.0 / (avg + jnp.float32(0.0001)) - 1.0
        iota_n = lax.broadcasted_iota(jnp.int32, (N, B), 0)
        for c in range(C):
            z = ht[c * N:(c + 1) * N, :] / temp + g_ref[c, :, :]
            m = jnp.max(z, axis=0, keepdims=True)
            cand = jnp.where(z == m, iota_n, N)
            am = jnp.min(cand, axis=0, keepdims=True)
            o_ref[c, :, :] = (iota_n == am).astype(jnp.int32)


def kernel(x, weights, input_filter):
    x4 = x.reshape(B, 1024, 1, CHUNK)
    wT = jnp.transpose(weights, (0, 2, 1))
    gT = _gumbel_cnb()

    xs4, row_sums = pl.pallas_call(
        _gather_body,
        grid_spec=pltpu.PrefetchScalarGridSpec(
            num_scalar_prefetch=1,
            grid=(J,),
            in_specs=[
                pl.BlockSpec((B, 1, 1, CHUNK),
                             lambda j, filt: (0, filt[j], 0, 0)),
            ],
            out_specs=[
                pl.BlockSpec((B, 1, 1, CHUNK), lambda j, filt: (0, j, 0, 0)),
                pl.BlockSpec((B, 1), lambda j, filt: (0, 0)),
            ],
            scratch_shapes=[pltpu.VMEM((B, 1), jnp.float32)],
        ),
        out_shape=[
            jax.ShapeDtypeStruct((B, J, 1, CHUNK), jnp.float32),
            jax.ShapeDtypeStruct((B, 1), jnp.float32),
        ],
    )(input_filter, x4)

    xs2 = xs4.reshape(B, K)

    out_cnb = pl.pallas_call(
        _main_body,
        grid=(T,),
        in_specs=[
            pl.BlockSpec((B, 1), lambda t: (0, 0)),
            pl.BlockSpec((B, BK), lambda t: (0, t)),
            pl.BlockSpec((C, N, BK), lambda t: (0, 0, t)),
            pl.BlockSpec((C, N, B), lambda t: (0, 0, 0)),
        ],
        out_specs=pl.BlockSpec((C, N, B), lambda t: (0, 0, 0)),
        scratch_shapes=[
            pltpu.VMEM((C * N, B), jnp.float32),
        ],
        out_shape=jax.ShapeDtypeStruct((C, N, B), jnp.int32),
    )(row_sums, xs2, wT, gT)

    return jnp.transpose(out_cnb, (2, 0, 1))


# speedup vs baseline: 6.4817x; 1.9532x over previous
import jax
import jax.numpy as jnp
import numpy as np
from jax import lax
from jax.experimental import pallas as pl
from jax.experimental.pallas import tpu as pltpu

B = 16
C = 32
N = 32
J = 64
M = 1024
CHUNK = 512
K = J * CHUNK
BK = 1024
T = K // BK


def _gumbel_cnb():
    g = jax.random.gumbel(jax.random.key(123), (B, C, N), jnp.float32)
    return jnp.transpose(g, (1, 2, 0))


def _gather_body(filt_ref, x_ref, o_ref, oht):
    b = pl.program_id(0)

    @pl.when(b == 0)
    def _():
        fcol = filt_ref[...]
        iota_m = lax.broadcasted_iota(jnp.int32, (J, M), 1)
        oht[...] = (iota_m == fcol).astype(jnp.float32)

    xb = x_ref[...]
    res = lax.dot_general(
        oht[...], xb, (((1,), (1,)), ((), ())),
        precision=jax.lax.Precision.HIGHEST,
        preferred_element_type=jnp.float32)
    o_ref[...] = res.reshape(1, J, 1, CHUNK)


def _main_body(xsfull_ref, xs_ref, w_ref, g_ref, o_ref, ht, sref):
    t = pl.program_id(0)

    @pl.when(t == 0)
    def _():
        ht[...] = jnp.zeros_like(ht)
        xf = xsfull_ref[:, :, 0, :]
        tmp = jnp.sum(xf, axis=1)
        sref[...] = jnp.sum(tmp, axis=1, keepdims=True)

    s = sref[...]
    wt2 = w_ref[...].reshape(C * N, BK)
    for jp in range(2):
        xq = xs_ref[:, jp, 0, :]
        xqn = jnp.where(s > 0.0, xq / s, 0.0)
        wtj = wt2[:, jp * CHUNK:(jp + 1) * CHUNK]
        ht[...] += lax.dot_general(
            wtj, xqn, (((1,), (1,)), ((), ())),
            preferred_element_type=jnp.float32)

    @pl.when(t == T - 1)
    def _():
        total = jnp.float32(0.0)
        for c in range(C):
            total += jnp.sum(jnp.max(ht[c * N:(c + 1) * N, :], axis=0))
        avg = total / jnp.float32(B * C)
        temp = 1.0 / (avg + jnp.float32(0.0001)) - 1.0
        iota_n = lax.broadcasted_iota(jnp.int32, (N, B), 0)
        for c in range(C):
            z = ht[c * N:(c + 1) * N, :] / temp + g_ref[c, :, :]
            m = jnp.max(z, axis=0, keepdims=True)
            cand = jnp.where(z == m, iota_n, N)
            am = jnp.min(cand, axis=0, keepdims=True)
            o_ref[c, :, :] = (iota_n == am).astype(jnp.int32)


def kernel(x, weights, input_filter):
    xflat = jnp.transpose(x, (0, 2, 3, 1)).reshape(B * CHUNK, M)
    wT = jnp.transpose(weights, (0, 2, 1))
    filt2 = input_filter.reshape(J, 1)
    gT = _gumbel_cnb()

    xsJ = pl.pallas_call(
        _gather_body,
        grid=(B,),
        in_specs=[
            pl.BlockSpec((J, 1), lambda b: (0, 0)),
            pl.BlockSpec((CHUNK, M), lambda b: (b, 0)),
        ],
        out_specs=pl.BlockSpec((1, J, 1, CHUNK), lambda b: (b, 0, 0, 0)),
        scratch_shapes=[pltpu.VMEM((J, M), jnp.float32)],
        out_shape=jax.ShapeDtypeStruct((B, J, 1, CHUNK), jnp.float32),
    )(filt2, xflat)

    out_cnb = pl.pallas_call(
        _main_body,
        grid=(T,),
        in_specs=[
            pl.BlockSpec((B, J, 1, CHUNK), lambda t: (0, 0, 0, 0)),
            pl.BlockSpec((B, 2, 1, CHUNK), lambda t: (0, t, 0, 0)),
            pl.BlockSpec((C, N, BK), lambda t: (0, 0, t)),
            pl.BlockSpec((C, N, B), lambda t: (0, 0, 0)),
        ],
        out_specs=pl.BlockSpec((C, N, B), lambda t: (0, 0, 0)),
        scratch_shapes=[
            pltpu.VMEM((C * N, B), jnp.float32),
            pltpu.VMEM((B, 1), jnp.float32),
        ],
        out_shape=jax.ShapeDtypeStruct((C, N, B), jnp.int32),
    )(xsJ, xsJ, wT, gT)

    return jnp.transpose(out_cnb, (2, 0, 1))
